# baseline (speedup 1.0000x reference)
import functools

import jax
import jax.numpy as jnp
from jax import lax
from jax.experimental import pallas as pl
from jax.experimental.pallas import tpu as pltpu
from jax.experimental.pallas import tpu_sc as plsc

K = 16
R = 256
CAP = 4176
BIG = 1 << 30


def _dist_block(q_ref, kt_ref, dist_ref, t_ref):
    q = q_ref[0]
    kt = kt_ref[0]
    cross = jnp.dot(q, kt, preferred_element_type=jnp.float32)
    q2 = jnp.sum(q * q, axis=1, keepdims=True)
    x2 = jnp.sum(kt * kt, axis=0, keepdims=True)
    d = (q2 + x2) - 2.0 * cross
    dist_ref[0] = d
    cm = d
    while cm.shape[1] > 128:
        h = cm.shape[1] // 2
        cm = jnp.minimum(cm[:, :h], cm[:, h:])
    col = lax.broadcasted_iota(jnp.int32, cm.shape, 1)
    for _ in range(K - 1):
        m = jnp.min(cm, axis=1, keepdims=True)
        ji = jnp.min(jnp.where(cm == m, col, jnp.int32(128)), axis=1,
                     keepdims=True)
        cm = jnp.where(col == ji, jnp.float32(jnp.inf), cm)
    t = jnp.min(cm, axis=1, keepdims=True)
    t_ref[0] = t


def _stage1(xyzp, kt):
    b, n, _ = xyzp.shape
    grid = (b, n // R)
    return pl.pallas_call(
        _dist_block,
        grid=grid,
        in_specs=[
            pl.BlockSpec((1, R, 8), lambda i, j: (i, j, 0)),
            pl.BlockSpec((1, 8, n), lambda i, j: (i, 0, 0)),
        ],
        out_specs=[
            pl.BlockSpec((1, R, n), lambda i, j: (i, j, 0)),
            pl.BlockSpec((1, R, 1), lambda i, j: (i, j, 0)),
        ],
        out_shape=[
            jax.ShapeDtypeStruct((b, n, n), jnp.float32),
            jax.ShapeDtypeStruct((b, n, 1), jnp.float32),
        ],
    )(xyzp, kt)


def _process_row(buf, tv, vcand, icand):
    lanes = lax.iota(jnp.int32, 16)

    def collect(j, ptr):
        d = buf[pl.ds(j * 16, 16)]
        mask = d <= tv
        cnt = plsc.all_reduce_population_count(mask)[0]

        def taken(p):
            kk = jnp.where(mask, d, jnp.float32(jnp.inf))
            sk, si = plsc.sort_key_val(kk, lanes + j * 16)
            vcand[pl.ds(p, 16)] = sk
            icand[pl.ds(p, 16)] = si
            return p + cnt

        return lax.cond(cnt > 0, taken, lambda p: p, ptr)

    ptr = lax.fori_loop(0, 256, collect, jnp.int32(0), unroll=2)
    vcand[pl.ds(ptr, 16)] = jnp.full((16,), jnp.inf, jnp.float32)
    icand[pl.ds(ptr, 16)] = jnp.full((16,), BIG, jnp.int32)
    nch = (ptr + 15) // 16

    def select_k(k, carry):
        gm, gi, acc = carry

        def scan_chunk(j, mc):
            mv, mi = mc
            v = vcand[pl.ds(j * 16, 16)]
            i = icand[pl.ds(j * 16, 16)]
            gt = (v > gm) | ((v == gm) & (i > gi))
            lt = gt & ((v < mv) | ((v == mv) & (i < mi)))
            return jnp.where(lt, v, mv), jnp.where(lt, i, mi)

        mv, mi = lax.fori_loop(
            0, nch, scan_chunk,
            (jnp.full((16,), jnp.inf, jnp.float32),
             jnp.full((16,), BIG, jnp.int32)))
        ngm = jnp.min(mv)
        ngi = jnp.min(jnp.where(mv == ngm, mi, BIG))
        acc = jnp.where(lanes == k, ngi, acc)
        return ngm, ngi, acc

    _, _, acc = lax.fori_loop(
        0, K, select_k,
        (jnp.float32(-jnp.inf), jnp.int32(-1), jnp.zeros((16,), jnp.int32)))
    return acc


def _sc_body(dist_hbm, t_hbm, out_hbm, bufa, bufb, tbuf, vcand, icand,
             obuf, sema, semb, semt):
    nc = 2
    wid = lax.axis_index("s") * nc + lax.axis_index("c")
    base = wid * 512

    pltpu.async_copy(t_hbm.at[pl.ds(base, 512)], tbuf, semt).wait()

    def issue(r, buf, sem):
        @pl.when(r < 512)
        def _():
            pltpu.async_copy(dist_hbm.at[base + r], buf, sem)

    def wait(buf, sem):
        pltpu.make_async_copy(dist_hbm.at[0], buf, sem).wait()

    issue(jnp.int32(0), bufa, sema)

    def group(g, _):
        tg = tbuf[pl.ds(g * 16, 16)]
        for i in range(16):
            r = g * 16 + i
            buf, sem = (bufa, sema) if i % 2 == 0 else (bufb, semb)
            nbuf, nsem = (bufb, semb) if i % 2 == 0 else (bufa, sema)
            wait(buf, sem)
            issue(r + 1, nbuf, nsem)
            tv = jnp.full((16,), tg[i], jnp.float32)
            obuf[r] = _process_row(buf, tv, vcand, icand)
        return 0

    lax.fori_loop(0, 32, group, 0)
    pltpu.sync_copy(obuf, out_hbm.at[pl.ds(base, 512)])


def _stage2(dist, t):
    mesh = plsc.VectorSubcoreMesh(core_axis_name="c", subcore_axis_name="s")
    nrows = dist.shape[0]
    f = pl.kernel(
        _sc_body,
        mesh=mesh,
        compiler_params=pltpu.CompilerParams(needs_layout_passes=False),
        out_type=jax.ShapeDtypeStruct((nrows, K), jnp.int32),
        scratch_types=[
            pltpu.VMEM((4096,), jnp.float32),
            pltpu.VMEM((4096,), jnp.float32),
            pltpu.VMEM((512,), jnp.float32),
            pltpu.VMEM((CAP,), jnp.float32),
            pltpu.VMEM((CAP,), jnp.int32),
            pltpu.VMEM((512, K), jnp.int32),
            pltpu.SemaphoreType.DMA,
            pltpu.SemaphoreType.DMA,
            pltpu.SemaphoreType.DMA,
        ],
    )
    return f(dist, t)


def kernel(xyz):
    b, n, _ = xyz.shape
    xyzp = jnp.pad(xyz, ((0, 0), (0, 0), (0, 5)))
    kt = xyzp.transpose(0, 2, 1)
    dist, t = _stage1(xyzp, kt)
    idx = _stage2(dist.reshape(b * n, n), t.reshape(b * n))
    return idx.reshape(b, n, K)


# speedup vs baseline: 2.4023x; 2.4023x over previous
import functools

import jax
import jax.numpy as jnp
from jax import lax
from jax.experimental import pallas as pl
from jax.experimental.pallas import tpu as pltpu
from jax.experimental.pallas import tpu_sc as plsc

K = 16
R = 256
LSTRIDE = 260
CAP = 16 * LSTRIDE
BIG = 1 << 30


def _dist_block(q_ref, kt_ref, dist_ref, t_ref):
    q = q_ref[0]
    kt = kt_ref[0]
    cross = jnp.dot(q, kt, preferred_element_type=jnp.float32)
    q2 = jnp.sum(q * q, axis=1, keepdims=True)
    x2 = jnp.sum(kt * kt, axis=0, keepdims=True)
    d = (q2 + x2) - 2.0 * cross
    dist_ref[0] = d
    cm = d
    while cm.shape[1] > 128:
        h = cm.shape[1] // 2
        cm = jnp.minimum(cm[:, :h], cm[:, h:])
    col = lax.broadcasted_iota(jnp.int32, cm.shape, 1)
    for _ in range(K - 1):
        m = jnp.min(cm, axis=1, keepdims=True)
        ji = jnp.min(jnp.where(cm == m, col, jnp.int32(128)), axis=1,
                     keepdims=True)
        cm = jnp.where(col == ji, jnp.float32(jnp.inf), cm)
    t = jnp.min(cm, axis=1, keepdims=True)
    t_ref[0] = t


def _stage1(xyzp, kt):
    b, n, _ = xyzp.shape
    grid = (b, n // R)
    return pl.pallas_call(
        _dist_block,
        grid=grid,
        in_specs=[
            pl.BlockSpec((1, R, 8), lambda i, j: (i, j, 0)),
            pl.BlockSpec((1, 8, n), lambda i, j: (i, 0, 0)),
        ],
        out_specs=[
            pl.BlockSpec((1, R, n), lambda i, j: (i, j, 0)),
            pl.BlockSpec((1, R, 1), lambda i, j: (i, j, 0)),
        ],
        out_shape=[
            jax.ShapeDtypeStruct((b, n, n), jnp.float32),
            jax.ShapeDtypeStruct((b, n, 1), jnp.float32),
        ],
    )(xyzp, kt)


def _process_row(buf, tv, vcand, icand, vcand2, icand2):
    lanes = lax.iota(jnp.int32, 16)
    lbase = lanes * LSTRIDE
    inf16 = jnp.full((16,), jnp.inf, jnp.float32)
    big16 = jnp.full((16,), BIG, jnp.int32)

    def collect(j, cnt):
        d = buf[pl.ds(j * 16, 16)]
        mask = d <= tv
        pos = lbase + cnt
        plsc.store_scatter(vcand, [pos], d, mask=mask)
        plsc.store_scatter(icand, [pos], jnp.full((16,), j * 16, jnp.int32),
                           mask=mask)
        return cnt + mask.astype(jnp.int32)

    cnt = lax.fori_loop(0, 256, collect, jnp.zeros((16,), jnp.int32),
                        unroll=4)
    plsc.store_scatter(vcand, [lbase + cnt], inf16)
    plsc.store_scatter(icand, [lbase + cnt], big16)
    maxc = jnp.max(cnt)

    def compact(j, p):
        m = j < cnt
        idx = lbase + jnp.minimum(cnt, j)
        v = plsc.load_gather(vcand, [idx])
        iv = plsc.load_gather(icand, [idx])
        v = jnp.where(m, v, jnp.float32(jnp.inf))
        iv = jnp.where(m, iv + lanes, BIG)
        sk, si = plsc.sort_key_val(v, iv)
        vcand2[pl.ds(p, 16)] = sk
        icand2[pl.ds(p, 16)] = si
        return p + plsc.all_reduce_population_count(m)[0]

    cptr = lax.fori_loop(0, maxc, compact, jnp.int32(0))
    vcand2[pl.ds(cptr, 16)] = inf16
    icand2[pl.ds(cptr, 16)] = big16
    nch = (cptr + 15) // 16

    def select_k(k, carry):
        gm, gi, acc = carry

        def scan_chunk(j, mc):
            mv, mi = mc
            v = vcand2[pl.ds(j * 16, 16)]
            i = icand2[pl.ds(j * 16, 16)]
            gt = (v > gm) | ((v == gm) & (i > gi))
            lt = gt & ((v < mv) | ((v == mv) & (i < mi)))
            return jnp.where(lt, v, mv), jnp.where(lt, i, mi)

        mv, mi = lax.fori_loop(
            0, nch, scan_chunk,
            (jnp.full((16,), jnp.inf, jnp.float32),
             jnp.full((16,), BIG, jnp.int32)))
        ngm = jnp.min(mv)
        ngi = jnp.min(jnp.where(mv == ngm, mi, BIG))
        acc = jnp.where(lanes == k, ngi, acc)
        return ngm, ngi, acc

    _, _, acc = lax.fori_loop(
        0, K, select_k,
        (jnp.float32(-jnp.inf), jnp.int32(-1), jnp.zeros((16,), jnp.int32)))
    return acc


def _sc_body(dist_hbm, t_hbm, out_hbm, bufa, bufb, tbuf, vcand, icand,
             vcand2, icand2, obuf, sema, semb, semt):
    nc = 2
    wid = lax.axis_index("s") * nc + lax.axis_index("c")
    base = wid * 512

    pltpu.async_copy(t_hbm.at[pl.ds(base, 512)], tbuf, semt).wait()

    def issue(r, buf, sem):
        @pl.when(r < 512)
        def _():
            pltpu.async_copy(dist_hbm.at[base + r], buf, sem)

    def wait(buf, sem):
        pltpu.make_async_copy(dist_hbm.at[0], buf, sem).wait()

    issue(jnp.int32(0), bufa, sema)

    def group(g, _):
        tg = tbuf[pl.ds(g * 16, 16)]
        for i in range(16):
            r = g * 16 + i
            buf, sem = (bufa, sema) if i % 2 == 0 else (bufb, semb)
            nbuf, nsem = (bufb, semb) if i % 2 == 0 else (bufa, sema)
            wait(buf, sem)
            issue(r + 1, nbuf, nsem)
            tv = jnp.full((16,), tg[i], jnp.float32)
            obuf[r] = _process_row(buf, tv, vcand, icand, vcand2, icand2)
        return 0

    lax.fori_loop(0, 32, group, 0)
    pltpu.sync_copy(obuf, out_hbm.at[pl.ds(base, 512)])


def _stage2(dist, t):
    mesh = plsc.VectorSubcoreMesh(core_axis_name="c", subcore_axis_name="s")
    nrows = dist.shape[0]
    f = pl.kernel(
        _sc_body,
        mesh=mesh,
        compiler_params=pltpu.CompilerParams(needs_layout_passes=False),
        out_type=jax.ShapeDtypeStruct((nrows, K), jnp.int32),
        scratch_types=[
            pltpu.VMEM((4096,), jnp.float32),
            pltpu.VMEM((4096,), jnp.float32),
            pltpu.VMEM((512,), jnp.float32),
            pltpu.VMEM((CAP,), jnp.float32),
            pltpu.VMEM((CAP,), jnp.int32),
            pltpu.VMEM((CAP,), jnp.float32),
            pltpu.VMEM((CAP,), jnp.int32),
            pltpu.VMEM((512, K), jnp.int32),
            pltpu.SemaphoreType.DMA,
            pltpu.SemaphoreType.DMA,
            pltpu.SemaphoreType.DMA,
        ],
    )
    return f(dist, t)


def kernel(xyz):
    b, n, _ = xyz.shape
    xyzp = jnp.pad(xyz, ((0, 0), (0, 0), (0, 5)))
    kt = xyzp.transpose(0, 2, 1)
    dist, t = _stage1(xyzp, kt)
    idx = _stage2(dist.reshape(b * n, n), t.reshape(b * n))
    return idx.reshape(b, n, K)
